# baseline (speedup 1.0000x reference)
import jax
import jax.numpy as jnp
from jax import lax
from jax.experimental import pallas as pl
from jax.experimental.pallas import tpu as pltpu
from jax.experimental.pallas import tpu_sc as plsc

N = 5000
NUM_CLASSES = 8
IOU_THRESHOLD = 0.5
L = 16
NIN = 5008
KCAP = 5120

_MESH = plsc.VectorSubcoreMesh(core_axis_name="c", subcore_axis_name="s",
                               num_cores=2, num_subcores=16)


def _lane_max(v, iota):
    for sh in (1, 2, 4, 8):
        v = jnp.maximum(v, jnp.take(v, iota ^ sh))
    return v


def _sc_nms_body(ycs_h, xcs_h, hs_h, ws_h, order_h, meta_h, out_h,
                 yc_v, xc_v, h_v, w_v, ord_v, meta_v,
                 ky1, kx1, ky2, kx2, kar, keep_v):
    cid = lax.axis_index("c")
    sid = lax.axis_index("s")

    @pl.when((cid == 0) & (sid < NUM_CLASSES))
    def _():
        wid = sid
        pltpu.sync_copy(ycs_h, yc_v)
        pltpu.sync_copy(xcs_h, xc_v)
        pltpu.sync_copy(hs_h, h_v)
        pltpu.sync_copy(ws_h, w_v)
        pltpu.sync_copy(order_h, ord_v)
        pltpu.sync_copy(meta_h, meta_v)

        iota = lax.iota(jnp.int32, L)
        widv = jnp.full((L,), wid, jnp.int32)
        start = plsc.load_gather(meta_v, [widv])[0].astype(jnp.int32)
        cnt = plsc.load_gather(meta_v, [widv + NUM_CLASSES])[0].astype(
            jnp.int32)

        def zero_body(i, carry):
            keep_v[pl.ds(i * L, L)] = jnp.zeros((L,), jnp.float32)
            return carry

        lax.fori_loop(0, NIN // L, zero_body, 0)

        sent1 = jnp.full((L,), 3.0e30, jnp.float32)
        sent2 = jnp.full((L,), -3.0e30, jnp.float32)
        zero = jnp.zeros((L,), jnp.float32)

        def sent_body(i, carry):
            sl = pl.ds(i * L, L)
            ky1[sl] = sent1
            kx1[sl] = sent1
            ky2[sl] = sent2
            kx2[sl] = sent2
            kar[sl] = zero
            return carry

        lax.fori_loop(0, KCAP // L, sent_body, 0)

        lane0 = iota == 0

        def fetch(gidx):
            oidx = plsc.load_gather(ord_v, [gidx]).astype(jnp.int32)
            ycc = plsc.load_gather(yc_v, [oidx])
            xcc = plsc.load_gather(xc_v, [oidx])
            hc = jnp.abs(plsc.load_gather(h_v, [oidx]))
            wc = jnp.abs(plsc.load_gather(w_v, [oidx]))
            return (ycc - 0.5 * hc, xcc - 0.5 * wc,
                    ycc + 0.5 * hc, xcc + 0.5 * wc, hc * wc)

        def iou_gt(y1a, x1a, y2a, x2a, aa, y1b, x1b, y2b, x2b, ab):
            iy1 = jnp.maximum(y1a, y1b)
            ix1 = jnp.maximum(x1a, x1b)
            iy2 = jnp.minimum(y2a, y2b)
            ix2 = jnp.minimum(x2a, x2b)
            ih = jnp.maximum(iy2 - iy1, 0.0)
            iw = jnp.maximum(ix2 - ix1, 0.0)
            inter = ih * iw
            union = aa + ab - inter
            iou = inter / jnp.maximum(union, 1e-9)
            return jnp.where(iou > IOU_THRESHOLD, 1.0, 0.0)

        npairs = (cnt + 1) // 2

        def cand2(p, kcnt):
            g0 = start + 2 * p
            gidx0 = jnp.full((L,), g0, jnp.int32)
            gidx1 = gidx0 + 1
            a = fetch(gidx0)
            b = fetch(gidx1)

            def sweep(sl, hits):
                hA, hB = hits
                k1 = ky1[sl]
                k2 = kx1[sl]
                k3 = ky2[sl]
                k4 = kx2[sl]
                k5 = kar[sl]
                hA = jnp.maximum(hA, iou_gt(k1, k2, k3, k4, k5, *a))
                hB = jnp.maximum(hB, iou_gt(k1, k2, k3, k4, k5, *b))
                return hA, hB

            nch4 = (kcnt + (4 * L - 1)) // (4 * L)

            def chunk(cix, hits):
                base = cix * (4 * L)
                hits = sweep(pl.ds(base, L), hits)
                hits = sweep(pl.ds(base + L, L), hits)
                hits = sweep(pl.ds(base + 2 * L, L), hits)
                return sweep(pl.ds(base + 3 * L, L), hits)

            z = jnp.zeros((L,), jnp.float32)
            hA, hB = lax.fori_loop(0, nch4, chunk, (z, z))

            supA = _lane_max(hA, iota)[0] > 0.5
            keptA = jnp.logical_not(supA)
            abhit = iou_gt(*a, *b)[0] > 0.5
            validB = (2 * p + 1) < cnt
            supB = (_lane_max(hB, iota)[0] > 0.5) | (keptA & abhit)
            keptB = jnp.logical_not(supB) & validB

            lane1 = iota == 1
            fAB = jnp.where(lane0, jnp.where(supA, 0.0, 1.0),
                            jnp.where(supB, 0.0, 1.0))
            plsc.store_scatter(keep_v, [gidx0 + iota], fAB,
                               mask=lane0 | (lane1 & validB))

            ia = jnp.where(keptA, 1, 0)
            kvec = jnp.full((L,), kcnt, jnp.int32) + jnp.where(lane1, ia, 0)
            amask = (lane0 & keptA) | (lane1 & keptB)
            plsc.store_scatter(ky1, [kvec], jnp.where(lane0, a[0], b[0]),
                               mask=amask)
            plsc.store_scatter(kx1, [kvec], jnp.where(lane0, a[1], b[1]),
                               mask=amask)
            plsc.store_scatter(ky2, [kvec], jnp.where(lane0, a[2], b[2]),
                               mask=amask)
            plsc.store_scatter(kx2, [kvec], jnp.where(lane0, a[3], b[3]),
                               mask=amask)
            plsc.store_scatter(kar, [kvec], jnp.where(lane0, a[4], b[4]),
                               mask=amask)
            return kcnt + ia + jnp.where(keptB, 1, 0)

        lax.fori_loop(0, npairs, cand2, jnp.int32(0))

        pltpu.sync_copy(keep_v, out_h.at[wid])


_sc_nms = pl.kernel(
    _sc_nms_body,
    out_type=jax.ShapeDtypeStruct((NUM_CLASSES, NIN), jnp.float32),
    mesh=_MESH,
    compiler_params=pltpu.CompilerParams(needs_layout_passes=False),
    scratch_types=[
        pltpu.VMEM((NIN,), jnp.float32),
        pltpu.VMEM((NIN,), jnp.float32),
        pltpu.VMEM((NIN,), jnp.float32),
        pltpu.VMEM((NIN,), jnp.float32),
        pltpu.VMEM((NIN,), jnp.float32),
        pltpu.VMEM((L,), jnp.float32),
        pltpu.VMEM((KCAP,), jnp.float32),
        pltpu.VMEM((KCAP,), jnp.float32),
        pltpu.VMEM((KCAP,), jnp.float32),
        pltpu.VMEM((KCAP,), jnp.float32),
        pltpu.VMEM((KCAP,), jnp.float32),
        pltpu.VMEM((NIN,), jnp.float32),
    ],
)


@jax.jit
def kernel(boxes, labels, scores):
    lab = labels.astype(jnp.int32)
    order = jnp.lexsort((-scores, lab))
    counts = jnp.zeros((NUM_CLASSES,), jnp.int32).at[lab].add(1)
    starts = jnp.concatenate([jnp.zeros((1,), jnp.int32),
                              jnp.cumsum(counts)[:-1].astype(jnp.int32)])
    meta = jnp.concatenate([starts, counts]).astype(jnp.float32)

    ordf = jnp.zeros((NIN,), jnp.float32).at[:N].set(order.astype(jnp.float32))

    def padded(col):
        return jnp.zeros((NIN,), jnp.float32).at[:N].set(col)

    bx = boxes.astype(jnp.float32)
    out8 = _sc_nms(padded(bx[:, 0]), padded(bx[:, 1]), padded(bx[:, 2]),
                   padded(bx[:, 3]), ordf, meta)

    keep_sorted = jnp.sum(out8, axis=0)[:N]
    m = jnp.zeros((N,), jnp.float32).at[order].set(keep_sorted)
    return jnp.concatenate([boxes * m[:, None], (scores * m)[:, None]], axis=1)


# speedup vs baseline: 1.0111x; 1.0111x over previous
import jax
import jax.numpy as jnp
from jax import lax
from jax.experimental import pallas as pl
from jax.experimental.pallas import tpu as pltpu
from jax.experimental.pallas import tpu_sc as plsc

N = 5000
NUM_CLASSES = 8
IOU_THRESHOLD = 0.5
L = 16
NIN = 5008
KCAP = 5120

_MESH = plsc.VectorSubcoreMesh(core_axis_name="c", subcore_axis_name="s",
                               num_cores=2, num_subcores=16)


def _lane_max(v, iota):
    for sh in (1, 2, 4, 8):
        v = jnp.maximum(v, jnp.take(v, iota ^ sh))
    return v


def _sc_nms_body(ycs_h, xcs_h, hs_h, ws_h, order_h, meta_h, out_h,
                 yc_v, xc_v, h_v, w_v, ord_v, meta_v,
                 ky1, kx1, ky2, kx2, kar, keep_v):
    cid = lax.axis_index("c")
    sid = lax.axis_index("s")

    @pl.when((cid == 0) & (sid < NUM_CLASSES))
    def _():
        wid = sid
        pltpu.sync_copy(ycs_h, yc_v)
        pltpu.sync_copy(xcs_h, xc_v)
        pltpu.sync_copy(hs_h, h_v)
        pltpu.sync_copy(ws_h, w_v)
        pltpu.sync_copy(order_h, ord_v)
        pltpu.sync_copy(meta_h, meta_v)

        iota = lax.iota(jnp.int32, L)
        widv = jnp.full((L,), wid, jnp.int32)
        start = plsc.load_gather(meta_v, [widv])[0].astype(jnp.int32)
        cnt = plsc.load_gather(meta_v, [widv + NUM_CLASSES])[0].astype(
            jnp.int32)

        def zero_body(i, carry):
            keep_v[pl.ds(i * L, L)] = jnp.zeros((L,), jnp.float32)
            return carry

        lax.fori_loop(0, NIN // L, zero_body, 0)

        sent1 = jnp.full((L,), 3.0e30, jnp.float32)
        sent2 = jnp.full((L,), -3.0e30, jnp.float32)
        zero = jnp.zeros((L,), jnp.float32)

        def sent_body(i, carry):
            sl = pl.ds(i * L, L)
            ky1[sl] = sent1
            kx1[sl] = sent1
            ky2[sl] = sent2
            kx2[sl] = sent2
            kar[sl] = zero
            return carry

        lax.fori_loop(0, KCAP // L, sent_body, 0)

        lane0 = iota == 0

        def fetch(gidx):
            oidx = plsc.load_gather(ord_v, [gidx]).astype(jnp.int32)
            ycc = plsc.load_gather(yc_v, [oidx])
            xcc = plsc.load_gather(xc_v, [oidx])
            hc = jnp.abs(plsc.load_gather(h_v, [oidx]))
            wc = jnp.abs(plsc.load_gather(w_v, [oidx]))
            return (ycc - 0.5 * hc, xcc - 0.5 * wc,
                    ycc + 0.5 * hc, xcc + 0.5 * wc, hc * wc)

        def iou_gt(y1a, x1a, y2a, x2a, aa, y1b, x1b, y2b, x2b, ab):
            iy1 = jnp.maximum(y1a, y1b)
            ix1 = jnp.maximum(x1a, x1b)
            iy2 = jnp.minimum(y2a, y2b)
            ix2 = jnp.minimum(x2a, x2b)
            ih = jnp.maximum(iy2 - iy1, 0.0)
            iw = jnp.maximum(ix2 - ix1, 0.0)
            inter = ih * iw
            union = aa + ab - inter
            iou = inter / jnp.maximum(union, 1e-9)
            return jnp.where(iou > IOU_THRESHOLD, 1.0, 0.0)

        npairs = (cnt + 1) // 2

        def cand2(p, kcnt):
            g0 = start + 2 * p
            gidx0 = jnp.full((L,), g0, jnp.int32)
            gidx1 = gidx0 + 1
            a = fetch(gidx0)
            b = fetch(gidx1)

            def sweep(sl, hits):
                hA, hB = hits
                k1 = ky1[sl]
                k2 = kx1[sl]
                k3 = ky2[sl]
                k4 = kx2[sl]
                k5 = kar[sl]
                hA = jnp.maximum(hA, iou_gt(k1, k2, k3, k4, k5, *a))
                hB = jnp.maximum(hB, iou_gt(k1, k2, k3, k4, k5, *b))
                return hA, hB

            nch2 = (kcnt + (2 * L - 1)) // (2 * L)

            def chunk(cix, hits):
                base = cix * (2 * L)
                hits = sweep(pl.ds(base, L), hits)
                return sweep(pl.ds(base + L, L), hits)

            z = jnp.zeros((L,), jnp.float32)
            hA, hB = lax.fori_loop(0, nch2, chunk, (z, z))

            supA = _lane_max(hA, iota)[0] > 0.5
            keptA = jnp.logical_not(supA)
            abhit = iou_gt(*a, *b)[0] > 0.5
            validB = (2 * p + 1) < cnt
            supB = (_lane_max(hB, iota)[0] > 0.5) | (keptA & abhit)
            keptB = jnp.logical_not(supB) & validB

            lane1 = iota == 1
            fAB = jnp.where(lane0, jnp.where(supA, 0.0, 1.0),
                            jnp.where(supB, 0.0, 1.0))
            plsc.store_scatter(keep_v, [gidx0 + iota], fAB,
                               mask=lane0 | (lane1 & validB))

            ia = jnp.where(keptA, 1, 0)
            kvec = jnp.full((L,), kcnt, jnp.int32) + jnp.where(lane1, ia, 0)
            amask = (lane0 & keptA) | (lane1 & keptB)
            plsc.store_scatter(ky1, [kvec], jnp.where(lane0, a[0], b[0]),
                               mask=amask)
            plsc.store_scatter(kx1, [kvec], jnp.where(lane0, a[1], b[1]),
                               mask=amask)
            plsc.store_scatter(ky2, [kvec], jnp.where(lane0, a[2], b[2]),
                               mask=amask)
            plsc.store_scatter(kx2, [kvec], jnp.where(lane0, a[3], b[3]),
                               mask=amask)
            plsc.store_scatter(kar, [kvec], jnp.where(lane0, a[4], b[4]),
                               mask=amask)
            return kcnt + ia + jnp.where(keptB, 1, 0)

        lax.fori_loop(0, npairs, cand2, jnp.int32(0))

        pltpu.sync_copy(keep_v, out_h.at[wid])


_sc_nms = pl.kernel(
    _sc_nms_body,
    out_type=jax.ShapeDtypeStruct((NUM_CLASSES, NIN), jnp.float32),
    mesh=_MESH,
    compiler_params=pltpu.CompilerParams(needs_layout_passes=False),
    scratch_types=[
        pltpu.VMEM((NIN,), jnp.float32),
        pltpu.VMEM((NIN,), jnp.float32),
        pltpu.VMEM((NIN,), jnp.float32),
        pltpu.VMEM((NIN,), jnp.float32),
        pltpu.VMEM((NIN,), jnp.float32),
        pltpu.VMEM((L,), jnp.float32),
        pltpu.VMEM((KCAP,), jnp.float32),
        pltpu.VMEM((KCAP,), jnp.float32),
        pltpu.VMEM((KCAP,), jnp.float32),
        pltpu.VMEM((KCAP,), jnp.float32),
        pltpu.VMEM((KCAP,), jnp.float32),
        pltpu.VMEM((NIN,), jnp.float32),
    ],
)


@jax.jit
def kernel(boxes, labels, scores):
    lab = labels.astype(jnp.int32)
    order = jnp.lexsort((-scores, lab))
    counts = jnp.zeros((NUM_CLASSES,), jnp.int32).at[lab].add(1)
    starts = jnp.concatenate([jnp.zeros((1,), jnp.int32),
                              jnp.cumsum(counts)[:-1].astype(jnp.int32)])
    meta = jnp.concatenate([starts, counts]).astype(jnp.float32)

    ordf = jnp.zeros((NIN,), jnp.float32).at[:N].set(order.astype(jnp.float32))

    def padded(col):
        return jnp.zeros((NIN,), jnp.float32).at[:N].set(col)

    bx = boxes.astype(jnp.float32)
    out8 = _sc_nms(padded(bx[:, 0]), padded(bx[:, 1]), padded(bx[:, 2]),
                   padded(bx[:, 3]), ordf, meta)

    keep_sorted = jnp.sum(out8, axis=0)[:N]
    m = jnp.zeros((N,), jnp.float32).at[order].set(keep_sorted)
    return jnp.concatenate([boxes * m[:, None], (scores * m)[:, None]], axis=1)
